# baseline (speedup 1.0000x reference)
import functools

import jax
import jax.numpy as jnp
import numpy as np
from jax import lax
from jax.experimental import pallas as pl
from jax.experimental.pallas import tpu as pltpu
from jax.experimental.pallas import tpu_sc as plsc

D_MODEL = 64
SEQ = 50
NC, NS = 2, 16
NW = NC * NS
GATHER = 100
GPAD = 128
CHUNK = 200
CB = CHUNK // SEQ
LANES = 16
PACK_BLK = 10000


def _pos_encoding(max_len, d_model):
    position = jnp.arange(max_len, dtype=jnp.float32)[:, None]
    div_term = jnp.exp(
        jnp.arange(0, d_model, 2, dtype=jnp.float32) * (-np.log(10000.0) / d_model)
    )
    pe = jnp.zeros((max_len, d_model), dtype=jnp.float32)
    pe = pe.at[:, 0::2].set(jnp.sin(position * div_term))
    pe = pe.at[:, 1::2].set(jnp.cos(position * div_term))
    return pe


def _pack_table(w):
    v = w.shape[0]
    nblk = v // 2 // PACK_BLK

    def body(lo_ref, hi_ref, out_ref):
        out_ref[:, :D_MODEL] = lo_ref[...]
        out_ref[:, D_MODEL:] = hi_ref[...]

    return pl.pallas_call(
        body,
        grid=(nblk,),
        in_specs=[
            pl.BlockSpec((PACK_BLK, D_MODEL), lambda i: (i, 0)),
            pl.BlockSpec((PACK_BLK, D_MODEL), lambda i: (i + nblk, 0)),
        ],
        out_specs=pl.BlockSpec((PACK_BLK, 2 * D_MODEL), lambda i: (i, 0)),
        out_shape=jax.ShapeDtypeStruct((v // 2, 2 * D_MODEL), jnp.float32),
    )(w, w)


@functools.partial(jax.jit, static_argnames=("batch", "seq"))
def _embed(idxpar, pe, wp, batch, seq):
    nchunks = (batch // NW) // CB
    ngather = 2 * nchunks

    mesh = plsc.VectorSubcoreMesh(
        core_axis_name="c", subcore_axis_name="s", num_cores=NC, num_subcores=NS
    )

    @functools.partial(
        pl.kernel,
        out_type=jax.ShapeDtypeStruct((batch, seq, D_MODEL), jnp.float32),
        mesh=mesh,
        scratch_types=[
            pltpu.VMEM((SEQ, D_MODEL), jnp.float32),
        ]
        + [pltpu.VMEM((16, GPAD), jnp.int32) for _ in range(4)]
        + [pltpu.VMEM((GATHER, 2 * D_MODEL), jnp.float32) for _ in range(4)]
        + [pltpu.VMEM((CB, SEQ, D_MODEL), jnp.float32) for _ in range(2)]
        + [pltpu.SemaphoreType.DMA for _ in range(6)],
    )
    def body(idx_hbm, pe_hbm, table_hbm, out_hbm, pe_v, *rest):
        ring = [rest[0:2], rest[2:4]]
        gbuf = [rest[4:6], rest[6:8]]
        obuf = [rest[8], rest[9]]
        isem = [rest[10], rest[11]]
        gsem = [rest[12], rest[13]]
        osem = [rest[14], rest[15]]
        wid = lax.axis_index("s") * NC + lax.axis_index("c")
        pltpu.sync_copy(pe_hbm, pe_v)
        obatch = wid * (CB * nchunks)

        def stage_idx(c, p):
            for h in range(2):
                pltpu.async_copy(idx_hbm.at[wid, 2 * c + h], ring[p][h], isem[p])

        def wait_idx(p):
            for h in range(2):
                pltpu.make_async_copy(
                    idx_hbm.at[wid, h], ring[p][h], isem[p]
                ).wait()

        def start_gathers(p):
            for h in range(2):
                pltpu.async_copy(
                    table_hbm.at[ring[p][h].at[0, pl.ds(0, GATHER)]],
                    gbuf[p][h],
                    gsem[p],
                )

        def wait_gathers(p):
            for h in range(2):
                pltpu.make_async_copy(
                    table_hbm.at[ring[p][h].at[0, pl.ds(0, GATHER)]],
                    gbuf[p][h],
                    gsem[p],
                ).wait()

        def wait_store(p):
            pltpu.make_async_copy(
                obuf[p], out_hbm.at[pl.ds(0, CB)], osem[p]
            ).wait()

        stage_idx(0, 0)
        stage_idx(1, 1)
        wait_idx(0)
        start_gathers(0)

        def cc_body(cc, _):
            for p in range(2):
                c = 2 * cc + p

                @pl.when(c + 1 < nchunks)
                def _():
                    wait_idx(1 - p)
                    start_gathers(1 - p)

                wait_gathers(p)

                @pl.when(c >= 2)
                def _():
                    wait_store(p)

                for bi in range(CB):
                    h = bi // 2
                    base_i = (bi % 2) * SEQ

                    def li_body(li, _):
                        i = base_i + li
                        prow = 2 + lax.div(i, 8)
                        pcol = lax.rem(i, 8) * LANES
                        pb = ring[p][h][prow, pl.ds(pcol, LANES)].astype(
                            jnp.float32
                        )
                        for j in range(D_MODEL // LANES):
                            sl = pl.ds(j * LANES, LANES)
                            slr = pl.ds(D_MODEL + j * LANES, LANES)
                            a = gbuf[p][h][i, sl]
                            b = gbuf[p][h][i, slr]
                            obuf[p][bi, li, sl] = (
                                a + pb * (b - a) + pe_v[li, sl]
                            )
                        return 0

                    lax.fori_loop(0, SEQ, li_body, 0)

                pltpu.async_copy(
                    obuf[p], out_hbm.at[pl.ds(obatch + CB * c, CB)], osem[p]
                )

                @pl.when(c + 2 < nchunks)
                def _():
                    stage_idx(c + 2, p)
            return 0

        lax.fori_loop(0, nchunks // 2, cc_body, 0)
        wait_store(0)
        wait_store(1)

    return body(idxpar, pe, wp)


def kernel(x, W):
    batch, seq = x.shape
    pe = _pos_encoding(seq, D_MODEL)
    nchunks = (batch // NW) // CB
    xw = x.reshape(NW, 2 * nchunks, GATHER)
    half = W.shape[0] // 2
    rows = jnp.pad(xw % half, ((0, 0), (0, 0), (0, GPAD - GATHER)))
    pars = xw // half
    pb = jnp.repeat(pars[..., None], LANES, axis=-1)
    pb = pb.reshape(NW, 2 * nchunks, GATHER * LANES)
    pb = jnp.pad(pb, ((0, 0), (0, 0), (0, 13 * GPAD - GATHER * LANES)))
    pbrows = pb.reshape(NW, 2 * nchunks, 13, GPAD)
    z = jnp.zeros((NW, 2 * nchunks, 1, GPAD), jnp.int32)
    idxpar = jnp.concatenate(
        [rows[:, :, None, :], z, pbrows, z], axis=2
    )
    wp = _pack_table(W)
    return _embed(idxpar, pe, wp, batch, seq)


# speedup vs baseline: 1.1923x; 1.1923x over previous
import functools

import jax
import jax.numpy as jnp
import numpy as np
from jax import lax
from jax.experimental import pallas as pl
from jax.experimental.pallas import tpu as pltpu
from jax.experimental.pallas import tpu_sc as plsc

D_MODEL = 64
SEQ = 50
NC, NS = 2, 16
NW = NC * NS
GATHER = 100
CHUNK = 200
RING = 4
OBUF = 2
LANES = 16
REPS = CHUNK // SEQ


def _pos_encoding(max_len, d_model):
    position = jnp.arange(max_len, dtype=jnp.float32)[:, None]
    div_term = jnp.exp(
        jnp.arange(0, d_model, 2, dtype=jnp.float32) * (-np.log(10000.0) / d_model)
    )
    pe = jnp.zeros((max_len, d_model), dtype=jnp.float32)
    pe = pe.at[:, 0::2].set(jnp.sin(position * div_term))
    pe = pe.at[:, 1::2].set(jnp.cos(position * div_term))
    return pe


@functools.partial(jax.jit, static_argnames=("batch", "seq"))
def _embed(idx3, pe, table, batch, seq):
    b_total = batch * seq
    bpw = b_total // NW
    nchunks = bpw // CHUNK
    ng = nchunks // RING

    mesh = plsc.VectorSubcoreMesh(
        core_axis_name="c", subcore_axis_name="s", num_cores=NC, num_subcores=NS
    )

    @functools.partial(
        pl.kernel,
        out_type=jax.ShapeDtypeStruct((b_total, D_MODEL), jnp.float32),
        mesh=mesh,
        compiler_params=pltpu.CompilerParams(use_tc_tiling_on_sc=False),
        scratch_types=[
            pltpu.VMEM((2 * nchunks, GATHER), jnp.int32),
            pltpu.VMEM((SEQ, D_MODEL), jnp.float32),
        ]
        + [pltpu.VMEM((CHUNK, D_MODEL), jnp.float32) for _ in range(RING)]
        + [pltpu.VMEM((CHUNK, D_MODEL), jnp.float32) for _ in range(OBUF)]
        + [pltpu.SemaphoreType.DMA for _ in range(RING + OBUF)],
    )
    def body(
        idx_hbm, pe_hbm, table_hbm, out_hbm, idx_v, pe_v,
        g0, g1, g2, g3, o0, o1, gs0, gs1, gs2, gs3, os0, os1,
    ):
        gbuf = [g0, g1, g2, g3]
        obuf = [o0, o1]
        gsem = [gs0, gs1, gs2, gs3]
        osem = [os0, os1]
        wid = lax.axis_index("s") * NC + lax.axis_index("c")
        pltpu.sync_copy(idx_hbm.at[wid], idx_v)
        pltpu.sync_copy(pe_hbm, pe_v)
        base = wid * bpw

        def start_gather(c, b):
            pltpu.async_copy(
                table_hbm.at[idx_v.at[2 * c]], gbuf[b].at[pl.ds(0, GATHER)], gsem[b]
            )
            pltpu.async_copy(
                table_hbm.at[idx_v.at[2 * c + 1]],
                gbuf[b].at[pl.ds(GATHER, GATHER)],
                gsem[b],
            )

        def wait_gather(b):
            pltpu.make_async_copy(
                table_hbm.at[pl.ds(0, CHUNK)], gbuf[b], gsem[b]
            ).wait()

        def wait_store(ob):
            pltpu.make_async_copy(
                obuf[ob], out_hbm.at[pl.ds(base, CHUNK)], osem[ob]
            ).wait()

        for b in range(RING):
            start_gather(b, b)

        def g_body(g, _):
            for b in range(RING):
                c = RING * g + b
                ob = b % OBUF
                wait_gather(b)

                def l_body(l, _):
                    for j in range(D_MODEL // LANES):
                        sl = pl.ds(j * LANES, LANES)
                        pe_vec = pe_v[l, sl]
                        for rep in range(REPS):
                            r = l + rep * SEQ
                            obuf[ob][r, sl] = gbuf[b][r, sl] + pe_vec
                    return 0

                lax.fori_loop(0, SEQ, l_body, 0)

                @pl.when(g < ng - 1)
                def _():
                    start_gather(c + RING, b)

                if b >= OBUF:
                    wait_store(ob)
                else:

                    @pl.when(g > 0)
                    def _():
                        wait_store(ob)

                pltpu.async_copy(
                    obuf[ob], out_hbm.at[pl.ds(base + c * CHUNK, CHUNK)], osem[ob]
                )
            return 0

        lax.fori_loop(0, ng, g_body, 0)
        wait_store(0)
        wait_store(1)

    return body(idx3, pe, table)


def kernel(x, W):
    batch, seq = x.shape
    pe = _pos_encoding(seq, D_MODEL)
    idx3 = x.reshape(NW, -1, GATHER)
    out = _embed(idx3, pe, W, batch, seq)
    return out.reshape(batch, seq, D_MODEL)
